# baseline (speedup 1.0000x reference)
import functools

import jax
import jax.numpy as jnp
from jax import lax
from jax.experimental import pallas as pl
from jax.experimental.pallas import tpu as pltpu
from jax.experimental.pallas import tpu_sc as plsc

_N = 10000
_D = 128
_E = 320000
_EPS = 1e-5

_NC = 2
_NS = 16
_NW = _NC * _NS
_K = 128
_CPW = 80
_CPB = 8
_NBLK = _CPW // _CPB
_EPAD = _NW * _CPW * _K
_NPAD = 10112
_RPS = _NPAD // _NS
_NB = 2

_sc_mesh = plsc.VectorSubcoreMesh(core_axis_name="c", subcore_axis_name="s")


def _deg_body(col_hbm, zeros_hbm, out_hbm, idx_v, ones_v, acc,
              ssem0, ssem1, ssem2, ssem3):
    ssems = [ssem0, ssem1, ssem2, ssem3]
    c = lax.axis_index("c")
    s = lax.axis_index("s")
    wid = c * _NS + s
    pltpu.sync_copy(zeros_hbm.at[pl.ds(s * _RPS, _RPS)],
                    acc.at[pl.ds(s * _RPS, _RPS)])
    lane = lax.broadcasted_iota(jnp.int32, (16,), 0)
    pat = jnp.where(lane == 0, 1.0, 0.0).astype(jnp.float32)
    zv = jnp.zeros((16,), jnp.float32)

    def fill(j, carry):
        ones_v[j, pl.ds(0, 16)] = pat
        for l in range(1, _D // 16):
            ones_v[j, pl.ds(l * 16, 16)] = zv
        return carry

    lax.fori_loop(0, _K, fill, 0)
    pltpu.sync_copy(col_hbm.at[wid], idx_v)
    plsc.subcore_barrier()

    descs = [None] * _CPW
    for j in range(_CPW):
        b = j % 4
        if j >= 4:
            descs[j - 4].wait()
        descs[j] = pltpu.async_copy(ones_v, acc.at[idx_v.at[j]],
                                    ssems[b], add=True)
    for j in range(max(0, _CPW - 4), _CPW):
        descs[j].wait()
    plsc.subcore_barrier()
    pltpu.sync_copy(acc.at[pl.ds(s * _RPS, _RPS)],
                    out_hbm.at[c, pl.ds(s * _RPS, _RPS)])


_deg_call = pl.kernel(
    _deg_body,
    out_type=jax.ShapeDtypeStruct((_NC, _NPAD, _D), jnp.float32),
    mesh=_sc_mesh,
    scratch_types=[
        pltpu.VMEM((_CPW, _K), jnp.int32),
        pltpu.VMEM((_K, _D), jnp.float32),
        pltpu.VMEM_SHARED((_NPAD, _D), jnp.float32),
        pltpu.SemaphoreType.DMA,
        pltpu.SemaphoreType.DMA,
        pltpu.SemaphoreType.DMA,
        pltpu.SemaphoreType.DMA,
    ],
)


def _agg_body(row_hbm, col_hbm, y_hbm, zeros_hbm, out_hbm,
              ir0, ir1, ic0, ic1, rows0, rows1, acc,
              irs0, irs1, ics0, ics1, gs0, gs1, ss0, ss1):
    irb = [ir0, ir1]
    icb = [ic0, ic1]
    rows = [rows0, rows1]
    irsem = [irs0, irs1]
    icsem = [ics0, ics1]
    gsem = [gs0, gs1]
    ssem = [ss0, ss1]
    c = lax.axis_index("c")
    s = lax.axis_index("s")
    wid = c * _NS + s
    pltpu.sync_copy(zeros_hbm.at[pl.ds(s * _RPS, _RPS)],
                    acc.at[pl.ds(s * _RPS, _RPS)])
    plsc.subcore_barrier()

    irdescs = [None] * _NBLK
    icdescs = [None] * _NBLK
    gdescs = [None] * _CPW
    sdescs = [None] * _CPW

    def ir_fetch(blk):
        irdescs[blk] = pltpu.async_copy(
            row_hbm.at[wid, pl.ds(blk * _CPB, _CPB)],
            irb[blk % 2], irsem[blk % 2])

    def ic_fetch(blk):
        icdescs[blk] = pltpu.async_copy(
            col_hbm.at[wid, pl.ds(blk * _CPB, _CPB)],
            icb[blk % 2], icsem[blk % 2])

    for blk in range(min(2, _NBLK)):
        ir_fetch(blk)
        ic_fetch(blk)

    for j in range(_CPW + 1):
        if j < _CPW:
            b = j % _NB
            if j >= _NB:
                sdescs[j - _NB].wait()
                if (j - _NB) % _CPB == _CPB - 1:
                    nblk = (j - _NB) // _CPB + 2
                    if nblk < _NBLK:
                        ic_fetch(nblk)
            if j % _CPB == 0:
                irdescs[j // _CPB].wait()
            gdescs[j] = pltpu.async_copy(
                y_hbm.at[irb[(j // _CPB) % 2].at[j % _CPB]], rows[b],
                gsem[b])
        if j >= 1:
            jj = j - 1
            gdescs[jj].wait()
            if jj % _CPB == _CPB - 1 and jj // _CPB + 2 < _NBLK:
                ir_fetch(jj // _CPB + 2)
            if jj % _CPB == 0:
                icdescs[jj // _CPB].wait()
            sdescs[jj] = pltpu.async_copy(
                rows[jj % _NB], acc.at[icb[(jj // _CPB) % 2].at[jj % _CPB]],
                ssem[jj % _NB], add=True)
    for j in range(max(0, _CPW - _NB), _CPW):
        sdescs[j].wait()
    plsc.subcore_barrier()
    pltpu.sync_copy(acc.at[pl.ds(s * _RPS, _RPS)],
                    out_hbm.at[c, pl.ds(s * _RPS, _RPS)])


_agg_call = pl.kernel(
    _agg_body,
    out_type=jax.ShapeDtypeStruct((_NC, _NPAD, _D), jnp.float32),
    mesh=_sc_mesh,
    scratch_types=(
        [pltpu.VMEM((_CPB, _K), jnp.int32)] * 4
        + [pltpu.VMEM((_K, _D), jnp.float32)] * _NB
        + [pltpu.VMEM_SHARED((_NPAD, _D), jnp.float32)]
        + [pltpu.SemaphoreType.DMA] * (4 + 2 * _NB)
    ),
)


def _prep_body(x_ref, w_ref, d0_ref, d1_ref, y_ref):
    deg = (jnp.sum(d0_ref[...], axis=1, keepdims=True)
           + jnp.sum(d1_ref[...], axis=1, keepdims=True) + 1.0)
    dis = lax.rsqrt(deg)
    xw = jnp.dot(x_ref[...], w_ref[...], preferred_element_type=jnp.float32)
    y_ref[...] = xw * dis


_prep_call = pl.pallas_call(
    _prep_body,
    out_shape=jax.ShapeDtypeStruct((_N, _D), jnp.float32),
)


def _post_body(x_ref, y_ref, a0_ref, a1_ref, d0_ref, d1_ref, bgcn_ref,
               g1_ref, b1_ref, wf1_ref, bf1_ref, wf2_ref, bf2_ref,
               g2_ref, b2_ref, out_ref):
    deg = (jnp.sum(d0_ref[...], axis=1, keepdims=True)
           + jnp.sum(d1_ref[...], axis=1, keepdims=True) + 1.0)
    dis = lax.rsqrt(deg)
    agg = a0_ref[...] + a1_ref[...] + y_ref[...]
    t = x_ref[...] + bgcn_ref[...] + dis * agg
    mean = jnp.mean(t, axis=0, keepdims=True)
    var = jnp.mean((t - mean) * (t - mean), axis=0, keepdims=True)
    h1 = g1_ref[...] * (t - mean) * lax.rsqrt(var + _EPS) + b1_ref[...]
    ff = jnp.maximum(
        jnp.dot(h1, wf1_ref[...], preferred_element_type=jnp.float32)
        + bf1_ref[...], 0.0)
    u = h1 + jnp.dot(ff, wf2_ref[...],
                     preferred_element_type=jnp.float32) + bf2_ref[...]
    mean2 = jnp.mean(u, axis=0, keepdims=True)
    var2 = jnp.mean((u - mean2) * (u - mean2), axis=0, keepdims=True)
    out_ref[...] = (g2_ref[...] * (u - mean2) * lax.rsqrt(var2 + _EPS)
                    + b2_ref[...])


_post_call = pl.pallas_call(
    _post_body,
    out_shape=jax.ShapeDtypeStruct((_N, _D), jnp.float32),
)


def kernel(x, edge_index, W_gcn, b_gcn, bn1_gamma, bn1_beta,
           W_ff1, b_ff1, W_ff2, b_ff2, bn2_gamma, bn2_beta):
    row = edge_index[0]
    col = edge_index[1]
    pad = _EPAD - _E
    sink = _N + (jnp.arange(pad, dtype=jnp.int32) % (_NPAD - _N))
    rowp = jnp.concatenate([row, jnp.zeros((pad,), jnp.int32)]).reshape(
        _NW, _CPW, _K)
    colp = jnp.concatenate([col, sink]).reshape(_NW, _CPW, _K)
    zeros_acc = jnp.zeros((_NPAD, _D), jnp.float32)

    degp = _deg_call(colp, zeros_acc)
    d0 = degp[0, :_N, :16]
    d1 = degp[1, :_N, :16]

    y = _prep_call(x, W_gcn, d0, d1)

    aggp = _agg_call(rowp, colp, y, zeros_acc)
    a0 = aggp[0, :_N]
    a1 = aggp[1, :_N]

    return _post_call(
        x, y, a0, a1, d0, d1,
        b_gcn.reshape(1, _D),
        bn1_gamma.reshape(1, _D), bn1_beta.reshape(1, _D),
        W_ff1, b_ff1.reshape(1, 2 * _D),
        W_ff2, b_ff2.reshape(1, _D),
        bn2_gamma.reshape(1, _D), bn2_beta.reshape(1, _D),
    )


# speedup vs baseline: 2.2789x; 2.2789x over previous
import functools

import jax
import jax.numpy as jnp
from jax import lax
from jax.experimental import pallas as pl
from jax.experimental.pallas import tpu as pltpu
from jax.experimental.pallas import tpu_sc as plsc

_N = 10000
_D = 128
_E = 320000
_EPS = 1e-5

_NC = 2
_NS = 16
_NW = _NC * _NS
_K = 128
_CPW = 80
_CPB = 8
_NBLK = _CPW // _CPB
_EPAD = _NW * _CPW * _K
_NPAD = 10112
_RPS = _NPAD // _NS
_NB = 2

_sc_mesh = plsc.VectorSubcoreMesh(core_axis_name="c", subcore_axis_name="s")


def _deg_body(col_hbm, zeros_hbm, out_hbm, idx_v, ones_v, acc,
              ssem0, ssem1, ssem2, ssem3):
    ssems = [ssem0, ssem1, ssem2, ssem3]
    c = lax.axis_index("c")
    s = lax.axis_index("s")
    wid = c * _NS + s
    pltpu.sync_copy(zeros_hbm.at[pl.ds(s * _RPS, _RPS)],
                    acc.at[pl.ds(s * _RPS, _RPS)])
    lane = lax.broadcasted_iota(jnp.int32, (16,), 0)
    pat = jnp.where(lane == 0, 1.0, 0.0).astype(jnp.float32)
    zv = jnp.zeros((16,), jnp.float32)

    def fill(j, carry):
        ones_v[j, pl.ds(0, 16)] = pat
        for l in range(1, _D // 16):
            ones_v[j, pl.ds(l * 16, 16)] = zv
        return carry

    lax.fori_loop(0, _K, fill, 0)
    pltpu.sync_copy(col_hbm.at[wid], idx_v)
    plsc.subcore_barrier()

    descs = [None] * _CPW
    for j in range(_CPW):
        b = j % 4
        if j >= 4:
            descs[j - 4].wait()
        descs[j] = pltpu.async_copy(ones_v, acc.at[idx_v.at[j]],
                                    ssems[b], add=True)
    for j in range(max(0, _CPW - 4), _CPW):
        descs[j].wait()
    plsc.subcore_barrier()
    pltpu.sync_copy(acc.at[pl.ds(s * _RPS, _RPS)],
                    out_hbm.at[c, pl.ds(s * _RPS, _RPS)])


_deg_call = pl.kernel(
    _deg_body,
    out_type=jax.ShapeDtypeStruct((_NC, _NPAD, _D), jnp.float32),
    mesh=_sc_mesh,
    scratch_types=[
        pltpu.VMEM((_CPW, _K), jnp.int32),
        pltpu.VMEM((_K, _D), jnp.float32),
        pltpu.VMEM_SHARED((_NPAD, _D), jnp.float32),
        pltpu.SemaphoreType.DMA,
        pltpu.SemaphoreType.DMA,
        pltpu.SemaphoreType.DMA,
        pltpu.SemaphoreType.DMA,
    ],
)


def _agg_body(row_hbm, col_hbm, y_hbm, zeros_hbm, out_hbm,
              ir0, ir1, ic0, ic1, rows0, rows1, acc,
              irs0, irs1, ics0, ics1, gs0, gs1, ss0, ss1):
    irb = [ir0, ir1]
    icb = [ic0, ic1]
    rows = [rows0, rows1]
    irsem = [irs0, irs1]
    icsem = [ics0, ics1]
    gsem = [gs0, gs1]
    ssem = [ss0, ss1]
    c = lax.axis_index("c")
    s = lax.axis_index("s")
    wid = c * _NS + s
    pltpu.sync_copy(zeros_hbm.at[pl.ds(s * _RPS, _RPS)],
                    acc.at[pl.ds(s * _RPS, _RPS)])
    plsc.subcore_barrier()

    irdescs = [None] * _NBLK
    icdescs = [None] * _NBLK
    gdescs = [None] * _CPW
    sdescs = [None] * _CPW

    def ir_fetch(blk):
        irdescs[blk] = pltpu.async_copy(
            row_hbm.at[wid, pl.ds(blk * _CPB, _CPB)],
            irb[blk % 2], irsem[blk % 2])

    def ic_fetch(blk):
        icdescs[blk] = pltpu.async_copy(
            col_hbm.at[wid, pl.ds(blk * _CPB, _CPB)],
            icb[blk % 2], icsem[blk % 2])

    for blk in range(min(2, _NBLK)):
        ir_fetch(blk)
        ic_fetch(blk)

    for j in range(_CPW + 1):
        if j < _CPW:
            b = j % _NB
            if j >= _NB:
                sdescs[j - _NB].wait()
                if (j - _NB) % _CPB == _CPB - 1:
                    nblk = (j - _NB) // _CPB + 2
                    if nblk < _NBLK:
                        ic_fetch(nblk)
            if j % _CPB == 0:
                irdescs[j // _CPB].wait()
            gdescs[j] = pltpu.async_copy(
                y_hbm.at[irb[(j // _CPB) % 2].at[j % _CPB]], rows[b],
                gsem[b])
        if j >= 1:
            jj = j - 1
            gdescs[jj].wait()
            if jj % _CPB == _CPB - 1 and jj // _CPB + 2 < _NBLK:
                ir_fetch(jj // _CPB + 2)
            if jj % _CPB == 0:
                icdescs[jj // _CPB].wait()
            sdescs[jj] = pltpu.async_copy(
                rows[jj % _NB], acc.at[icb[(jj // _CPB) % 2].at[jj % _CPB]],
                ssem[jj % _NB], add=True)
    for j in range(max(0, _CPW - _NB), _CPW):
        sdescs[j].wait()
    plsc.subcore_barrier()
    pltpu.sync_copy(acc.at[pl.ds(s * _RPS, _RPS)],
                    out_hbm.at[c, pl.ds(s * _RPS, _RPS)])


_agg_call = pl.kernel(
    _agg_body,
    out_type=jax.ShapeDtypeStruct((_NC, _NPAD, _D), jnp.float32),
    mesh=_sc_mesh,
    scratch_types=(
        [pltpu.VMEM((_CPB, _K), jnp.int32)] * 4
        + [pltpu.VMEM((_K, _D), jnp.float32)] * _NB
        + [pltpu.VMEM_SHARED((_NPAD, _D), jnp.float32)]
        + [pltpu.SemaphoreType.DMA] * (4 + 2 * _NB)
    ),
)


def _xw_body(x_ref, w_ref, xw_ref):
    xw_ref[...] = jnp.dot(x_ref[...], w_ref[...],
                          preferred_element_type=jnp.float32)


_xw_call = pl.pallas_call(
    _xw_body,
    out_shape=jax.ShapeDtypeStruct((_N, _D), jnp.float32),
)


def _scale_body(xw_ref, d0_ref, d1_ref, y_ref):
    deg = (jnp.sum(d0_ref[...], axis=1, keepdims=True)
           + jnp.sum(d1_ref[...], axis=1, keepdims=True) + 1.0)
    dis = lax.rsqrt(deg)
    y_ref[...] = xw_ref[...] * dis


_scale_call = pl.pallas_call(
    _scale_body,
    out_shape=jax.ShapeDtypeStruct((_N, _D), jnp.float32),
)


def _post_body(x_ref, y_ref, a0_ref, a1_ref, d0_ref, d1_ref, bgcn_ref,
               g1_ref, b1_ref, wf1_ref, bf1_ref, wf2_ref, bf2_ref,
               g2_ref, b2_ref, out_ref):
    deg = (jnp.sum(d0_ref[...], axis=1, keepdims=True)
           + jnp.sum(d1_ref[...], axis=1, keepdims=True) + 1.0)
    dis = lax.rsqrt(deg)
    agg = a0_ref[...] + a1_ref[...] + y_ref[...]
    t = x_ref[...] + bgcn_ref[...] + dis * agg
    mean = jnp.mean(t, axis=0, keepdims=True)
    var = jnp.mean((t - mean) * (t - mean), axis=0, keepdims=True)
    h1 = g1_ref[...] * (t - mean) * lax.rsqrt(var + _EPS) + b1_ref[...]
    ff = jnp.maximum(
        jnp.dot(h1, wf1_ref[...], preferred_element_type=jnp.float32)
        + bf1_ref[...], 0.0)
    u = h1 + jnp.dot(ff, wf2_ref[...],
                     preferred_element_type=jnp.float32) + bf2_ref[...]
    mean2 = jnp.mean(u, axis=0, keepdims=True)
    var2 = jnp.mean((u - mean2) * (u - mean2), axis=0, keepdims=True)
    out_ref[...] = (g2_ref[...] * (u - mean2) * lax.rsqrt(var2 + _EPS)
                    + b2_ref[...])


_post_call = pl.pallas_call(
    _post_body,
    out_shape=jax.ShapeDtypeStruct((_N, _D), jnp.float32),
)


def kernel(x, edge_index, W_gcn, b_gcn, bn1_gamma, bn1_beta,
           W_ff1, b_ff1, W_ff2, b_ff2, bn2_gamma, bn2_beta):
    row = edge_index[0]
    col = edge_index[1]
    pad = _EPAD - _E
    sink = _N + (jnp.arange(pad, dtype=jnp.int32) % (_NPAD - _N))
    spread = jnp.arange(pad, dtype=jnp.int32) * 53 % _N
    rowp = jnp.concatenate([row, spread]).reshape(_NW, _CPW, _K)
    colp = jnp.concatenate([col, sink]).reshape(_NW, _CPW, _K)
    zeros_acc = jnp.zeros((_NPAD, _D), jnp.float32)

    xw = _xw_call(x, W_gcn)
    degp = _deg_call(colp, zeros_acc)
    d0 = degp[0, :_N, :16]
    d1 = degp[1, :_N, :16]

    y = _scale_call(xw, d0, d1)

    aggp = _agg_call(rowp, colp, y, zeros_acc)
    a0 = aggp[0, :_N]
    a1 = aggp[1, :_N]

    return _post_call(
        x, y, a0, a1, d0, d1,
        b_gcn.reshape(1, _D),
        bn1_gamma.reshape(1, _D), bn1_beta.reshape(1, _D),
        W_ff1, b_ff1.reshape(1, 2 * _D),
        W_ff2, b_ff2.reshape(1, _D),
        bn2_gamma.reshape(1, _D), bn2_beta.reshape(1, _D),
    )


# speedup vs baseline: 2.4062x; 1.0559x over previous
import functools

import jax
import jax.numpy as jnp
from jax import lax
from jax.experimental import pallas as pl
from jax.experimental.pallas import tpu as pltpu
from jax.experimental.pallas import tpu_sc as plsc

_N = 10000
_D = 128
_E = 320000
_EPS = 1e-5

_NC = 2
_NS = 16
_NW = _NC * _NS
_K = 128
_CPW = 80
_CPB = 8
_NBLK = _CPW // _CPB
_EPAD = _NW * _CPW * _K
_NPAD = 10112
_RPS = _NPAD // _NS
_NB = 2

_sc_mesh = plsc.VectorSubcoreMesh(core_axis_name="c", subcore_axis_name="s")


def _deg_body(col_hbm, zeros_hbm, out_hbm, idx_v, ones_v, acc,
              ssem0, ssem1, ssem2, ssem3):
    ssems = [ssem0, ssem1, ssem2, ssem3]
    c = lax.axis_index("c")
    s = lax.axis_index("s")
    wid = c * _NS + s
    pltpu.sync_copy(zeros_hbm.at[pl.ds(s * _RPS, _RPS)],
                    acc.at[pl.ds(s * _RPS, _RPS)])
    lane = lax.broadcasted_iota(jnp.int32, (16,), 0)
    pat = jnp.where(lane == 0, 1.0, 0.0).astype(jnp.float32)
    zv = jnp.zeros((16,), jnp.float32)

    def fill(j, carry):
        ones_v[j, pl.ds(0, 16)] = pat
        for l in range(1, _D // 16):
            ones_v[j, pl.ds(l * 16, 16)] = zv
        return carry

    lax.fori_loop(0, _K, fill, 0)
    pltpu.sync_copy(col_hbm.at[wid], idx_v)
    plsc.subcore_barrier()

    descs = [None] * _CPW
    for j in range(_CPW):
        b = j % 4
        if j >= 4:
            descs[j - 4].wait()
        descs[j] = pltpu.async_copy(ones_v, acc.at[idx_v.at[j]],
                                    ssems[b], add=True)
    for j in range(max(0, _CPW - 4), _CPW):
        descs[j].wait()
    plsc.subcore_barrier()
    pltpu.sync_copy(acc.at[pl.ds(s * _RPS, _RPS)],
                    out_hbm.at[c, pl.ds(s * _RPS, _RPS)])


_deg_call = pl.kernel(
    _deg_body,
    out_type=jax.ShapeDtypeStruct((_NC, _NPAD, _D), jnp.float32),
    mesh=_sc_mesh,
    scratch_types=[
        pltpu.VMEM((_CPW, _K), jnp.int32),
        pltpu.VMEM((_K, _D), jnp.float32),
        pltpu.VMEM_SHARED((_NPAD, _D), jnp.float32),
        pltpu.SemaphoreType.DMA,
        pltpu.SemaphoreType.DMA,
        pltpu.SemaphoreType.DMA,
        pltpu.SemaphoreType.DMA,
    ],
)


def _agg_body(row_hbm, col_hbm, y_hbm, zeros_hbm, out_hbm,
              ir0, ir1, ic0, ic1, rows0, rows1, acc,
              irs0, irs1, ics0, ics1, gs0, gs1, ss0, ss1):
    irb = [ir0, ir1]
    icb = [ic0, ic1]
    rows = [rows0, rows1]
    irsem = [irs0, irs1]
    icsem = [ics0, ics1]
    gsem = [gs0, gs1]
    ssem = [ss0, ss1]
    c = lax.axis_index("c")
    s = lax.axis_index("s")
    wid = c * _NS + s
    pltpu.sync_copy(zeros_hbm.at[pl.ds(s * _RPS, _RPS)],
                    acc.at[pl.ds(s * _RPS, _RPS)])
    plsc.subcore_barrier()

    irdescs = [None] * _NBLK
    icdescs = [None] * _NBLK
    gdescs = [None] * _CPW
    sdescs = [None] * _CPW

    def ir_fetch(blk):
        irdescs[blk] = pltpu.async_copy(
            row_hbm.at[wid, pl.ds(blk * _CPB, _CPB)],
            irb[blk % 2], irsem[blk % 2])

    def ic_fetch(blk):
        icdescs[blk] = pltpu.async_copy(
            col_hbm.at[wid, pl.ds(blk * _CPB, _CPB)],
            icb[blk % 2], icsem[blk % 2])

    for blk in range(min(2, _NBLK)):
        ir_fetch(blk)
        ic_fetch(blk)

    for j in range(_CPW + 1):
        if j < _CPW:
            b = j % _NB
            if j >= _NB:
                sdescs[j - _NB].wait()
                if (j - _NB) % _CPB == _CPB - 1:
                    nblk = (j - _NB) // _CPB + 2
                    if nblk < _NBLK:
                        ic_fetch(nblk)
            if j % _CPB == 0:
                irdescs[j // _CPB].wait()
            gdescs[j] = pltpu.async_copy(
                y_hbm.at[irb[(j // _CPB) % 2].at[j % _CPB]], rows[b],
                gsem[b])
        if j >= 1:
            jj = j - 1
            gdescs[jj].wait()
            if jj % _CPB == _CPB - 1 and jj // _CPB + 2 < _NBLK:
                ir_fetch(jj // _CPB + 2)
            if jj % _CPB == 0:
                icdescs[jj // _CPB].wait()
            sdescs[jj] = pltpu.async_copy(
                rows[jj % _NB], acc.at[icb[(jj // _CPB) % 2].at[jj % _CPB]],
                ssem[jj % _NB], add=True)
    for j in range(max(0, _CPW - _NB), _CPW):
        sdescs[j].wait()
    plsc.subcore_barrier()
    pltpu.sync_copy(acc.at[pl.ds(s * _RPS, _RPS)],
                    out_hbm.at[c, pl.ds(s * _RPS, _RPS)])


_agg_call = pl.kernel(
    _agg_body,
    out_type=jax.ShapeDtypeStruct((_NC, _NPAD, _D), jnp.float32),
    mesh=_sc_mesh,
    scratch_types=(
        [pltpu.VMEM((_CPB, _K), jnp.int32)] * 4
        + [pltpu.VMEM((_K, _D), jnp.float32)] * _NB
        + [pltpu.VMEM_SHARED((_NPAD, _D), jnp.float32)]
        + [pltpu.SemaphoreType.DMA] * (4 + 2 * _NB)
    ),
)


def _xw_body(x_ref, w_ref, xw_ref):
    xw_ref[...] = jnp.dot(x_ref[...], w_ref[...],
                          preferred_element_type=jnp.float32)


_xw_call = pl.pallas_call(
    _xw_body,
    out_shape=jax.ShapeDtypeStruct((_N, _D), jnp.float32),
)


def _scale_body(xw_ref, degp_ref, y_ref):
    dv = degp_ref[...]
    deg = (jnp.sum(dv[0, :_N, :16], axis=1, keepdims=True)
           + jnp.sum(dv[1, :_N, :16], axis=1, keepdims=True) + 1.0)
    dis = lax.rsqrt(deg)
    y_ref[...] = xw_ref[...] * dis


_scale_call = pl.pallas_call(
    _scale_body,
    out_shape=jax.ShapeDtypeStruct((_N, _D), jnp.float32),
)


def _post_body(x_ref, y_ref, aggp_ref, degp_ref, bgcn_ref,
               g1_ref, b1_ref, wf1_ref, bf1_ref, wf2_ref, bf2_ref,
               g2_ref, b2_ref, out_ref):
    dv = degp_ref[...]
    deg = (jnp.sum(dv[0, :_N, :16], axis=1, keepdims=True)
           + jnp.sum(dv[1, :_N, :16], axis=1, keepdims=True) + 1.0)
    dis = lax.rsqrt(deg)
    av = aggp_ref[...]
    agg = av[0, :_N] + av[1, :_N] + y_ref[...]
    t = x_ref[...] + bgcn_ref[...] + dis * agg
    mean = jnp.mean(t, axis=0, keepdims=True)
    var = jnp.mean((t - mean) * (t - mean), axis=0, keepdims=True)
    h1 = g1_ref[...] * (t - mean) * lax.rsqrt(var + _EPS) + b1_ref[...]
    ff = jnp.maximum(
        jnp.dot(h1, wf1_ref[...], preferred_element_type=jnp.float32)
        + bf1_ref[...], 0.0)
    u = h1 + jnp.dot(ff, wf2_ref[...],
                     preferred_element_type=jnp.float32) + bf2_ref[...]
    mean2 = jnp.mean(u, axis=0, keepdims=True)
    var2 = jnp.mean((u - mean2) * (u - mean2), axis=0, keepdims=True)
    out_ref[...] = (g2_ref[...] * (u - mean2) * lax.rsqrt(var2 + _EPS)
                    + b2_ref[...])


_post_call = pl.pallas_call(
    _post_body,
    out_shape=jax.ShapeDtypeStruct((_N, _D), jnp.float32),
)


def kernel(x, edge_index, W_gcn, b_gcn, bn1_gamma, bn1_beta,
           W_ff1, b_ff1, W_ff2, b_ff2, bn2_gamma, bn2_beta):
    row = edge_index[0]
    col = edge_index[1]
    pad = _EPAD - _E
    sink = _N + (jnp.arange(pad, dtype=jnp.int32) % (_NPAD - _N))
    spread = jnp.arange(pad, dtype=jnp.int32) * 53 % _N
    rowp = jnp.concatenate([row, spread]).reshape(_NW, _CPW, _K)
    colp = jnp.concatenate([col, sink]).reshape(_NW, _CPW, _K)
    zeros_acc = jnp.zeros((_NPAD, _D), jnp.float32)

    xw = _xw_call(x, W_gcn)
    degp = _deg_call(colp, zeros_acc)

    y = _scale_call(xw, degp)

    aggp = _agg_call(rowp, colp, y, zeros_acc)

    return _post_call(
        x, y, aggp, degp,
        b_gcn.reshape(1, _D),
        bn1_gamma.reshape(1, _D), bn1_beta.reshape(1, _D),
        W_ff1, b_ff1.reshape(1, 2 * _D),
        W_ff2, b_ff2.reshape(1, _D),
        bn2_gamma.reshape(1, _D), bn2_beta.reshape(1, _D),
    )


# speedup vs baseline: 2.4189x; 1.0053x over previous
import functools

import jax
import jax.numpy as jnp
from jax import lax
from jax.experimental import pallas as pl
from jax.experimental.pallas import tpu as pltpu
from jax.experimental.pallas import tpu_sc as plsc

_N = 10000
_D = 128
_E = 320000
_EPS = 1e-5

_NC = 2
_NS = 16
_NW = _NC * _NS
_K = 128
_CPW = 80
_CPB = 8
_NBLK = _CPW // _CPB
_EPAD = _NW * _CPW * _K
_NPAD = 10112
_RPS = _NPAD // _NS
_NB = 2

_sc_mesh = plsc.VectorSubcoreMesh(core_axis_name="c", subcore_axis_name="s")


def _deg_body(col_hbm, zeros_hbm, out_hbm, idx_v, ones_v, acc,
              ssem0, ssem1, ssem2, ssem3):
    ssems = [ssem0, ssem1, ssem2, ssem3]
    c = lax.axis_index("c")
    s = lax.axis_index("s")
    wid = c * _NS + s
    pltpu.sync_copy(zeros_hbm.at[pl.ds(s * _RPS, _RPS)],
                    acc.at[pl.ds(s * _RPS, _RPS)])
    lane = lax.broadcasted_iota(jnp.int32, (16,), 0)
    pat = jnp.where(lane == 0, 1.0, 0.0).astype(jnp.float32)
    zv = jnp.zeros((16,), jnp.float32)

    def fill(j, carry):
        ones_v[j, pl.ds(0, 16)] = pat
        for l in range(1, _D // 16):
            ones_v[j, pl.ds(l * 16, 16)] = zv
        return carry

    lax.fori_loop(0, _K, fill, 0)
    pltpu.sync_copy(col_hbm.at[wid], idx_v)
    plsc.subcore_barrier()

    descs = [None] * _CPW
    for j in range(_CPW):
        b = j % 4
        if j >= 4:
            descs[j - 4].wait()
        descs[j] = pltpu.async_copy(ones_v, acc.at[idx_v.at[j]],
                                    ssems[b], add=True)
    for j in range(max(0, _CPW - 4), _CPW):
        descs[j].wait()
    plsc.subcore_barrier()
    pltpu.sync_copy(acc.at[pl.ds(s * _RPS, _RPS)],
                    out_hbm.at[c, pl.ds(s * _RPS, _RPS)])


_deg_call = pl.kernel(
    _deg_body,
    out_type=jax.ShapeDtypeStruct((_NC, _NPAD, _D), jnp.float32),
    mesh=_sc_mesh,
    scratch_types=[
        pltpu.VMEM((_CPW, _K), jnp.int32),
        pltpu.VMEM((_K, _D), jnp.float32),
        pltpu.VMEM_SHARED((_NPAD, _D), jnp.float32),
        pltpu.SemaphoreType.DMA,
        pltpu.SemaphoreType.DMA,
        pltpu.SemaphoreType.DMA,
        pltpu.SemaphoreType.DMA,
    ],
)


def _agg_body(row_hbm, col_hbm, y_hbm, zeros_hbm, out_hbm,
              ir0, ir1, ic0, ic1, rows0, rows1, acc,
              irs0, irs1, ics0, ics1, gs0, gs1, ss0, ss1):
    irb = [ir0, ir1]
    icb = [ic0, ic1]
    rows = [rows0, rows1]
    irsem = [irs0, irs1]
    icsem = [ics0, ics1]
    gsem = [gs0, gs1]
    ssem = [ss0, ss1]
    c = lax.axis_index("c")
    s = lax.axis_index("s")
    wid = c * _NS + s
    pltpu.sync_copy(zeros_hbm.at[pl.ds(s * _RPS, _RPS)],
                    acc.at[pl.ds(s * _RPS, _RPS)])
    plsc.subcore_barrier()

    irdescs = [None] * _NBLK
    icdescs = [None] * _NBLK
    gdescs = [None] * _CPW
    sdescs = [None] * _CPW

    def ir_fetch(blk):
        irdescs[blk] = pltpu.async_copy(
            row_hbm.at[wid, pl.ds(blk * _CPB, _CPB)],
            irb[blk % 2], irsem[blk % 2])

    def ic_fetch(blk):
        icdescs[blk] = pltpu.async_copy(
            col_hbm.at[wid, pl.ds(blk * _CPB, _CPB)],
            icb[blk % 2], icsem[blk % 2])

    for blk in range(min(2, _NBLK)):
        ir_fetch(blk)
        ic_fetch(blk)

    for j in range(_CPW + 1):
        if j < _CPW:
            b = j % _NB
            if j >= _NB:
                sdescs[j - _NB].wait()
                if (j - _NB) % _CPB == _CPB - 1:
                    nblk = (j - _NB) // _CPB + 2
                    if nblk < _NBLK:
                        ic_fetch(nblk)
            if j % _CPB == 0:
                irdescs[j // _CPB].wait()
            gdescs[j] = pltpu.async_copy(
                y_hbm.at[irb[(j // _CPB) % 2].at[j % _CPB]], rows[b],
                gsem[b])
        if j >= 1:
            jj = j - 1
            gdescs[jj].wait()
            if jj % _CPB == _CPB - 1 and jj // _CPB + 2 < _NBLK:
                ir_fetch(jj // _CPB + 2)
            if jj % _CPB == 0:
                icdescs[jj // _CPB].wait()
            sdescs[jj] = pltpu.async_copy(
                rows[jj % _NB], acc.at[icb[(jj // _CPB) % 2].at[jj % _CPB]],
                ssem[jj % _NB], add=True)
    for j in range(max(0, _CPW - _NB), _CPW):
        sdescs[j].wait()
    plsc.subcore_barrier()
    pltpu.sync_copy(acc.at[pl.ds(s * _RPS, _RPS)],
                    out_hbm.at[c, pl.ds(s * _RPS, _RPS)])


_agg_call = pl.kernel(
    _agg_body,
    out_type=jax.ShapeDtypeStruct((_NC, _NPAD, _D), jnp.float32),
    mesh=_sc_mesh,
    scratch_types=(
        [pltpu.VMEM((_CPB, _K), jnp.int32)] * 4
        + [pltpu.VMEM((_K, _D), jnp.float32)] * _NB
        + [pltpu.VMEM_SHARED((_NPAD, _D), jnp.float32)]
        + [pltpu.SemaphoreType.DMA] * (4 + 2 * _NB)
    ),
)


def _xw_body(x_ref, w_ref, xw_ref):
    xw_ref[...] = jnp.dot(x_ref[...], w_ref[...],
                          preferred_element_type=jnp.float32)


_xw_call = pl.pallas_call(
    _xw_body,
    out_shape=jax.ShapeDtypeStruct((_N, _D), jnp.float32),
)


def _scale_body(x_ref, w_ref, degp_ref, y_ref):
    dv = degp_ref[...]
    deg = (jnp.sum(dv[0, :_N, :16], axis=1, keepdims=True)
           + jnp.sum(dv[1, :_N, :16], axis=1, keepdims=True) + 1.0)
    dis = lax.rsqrt(deg)
    xw = jnp.dot(x_ref[...], w_ref[...], preferred_element_type=jnp.float32)
    y_ref[...] = xw * dis


_scale_call = pl.pallas_call(
    _scale_body,
    out_shape=jax.ShapeDtypeStruct((_N, _D), jnp.float32),
)


def _post_body(x_ref, y_ref, aggp_ref, degp_ref, bgcn_ref,
               g1_ref, b1_ref, wf1_ref, bf1_ref, wf2_ref, bf2_ref,
               g2_ref, b2_ref, out_ref):
    dv = degp_ref[...]
    deg = (jnp.sum(dv[0, :_N, :16], axis=1, keepdims=True)
           + jnp.sum(dv[1, :_N, :16], axis=1, keepdims=True) + 1.0)
    dis = lax.rsqrt(deg)
    av = aggp_ref[...]
    agg = av[0, :_N] + av[1, :_N] + y_ref[...]
    t = x_ref[...] + bgcn_ref[...] + dis * agg
    mean = jnp.mean(t, axis=0, keepdims=True)
    var = jnp.mean((t - mean) * (t - mean), axis=0, keepdims=True)
    h1 = g1_ref[...] * (t - mean) * lax.rsqrt(var + _EPS) + b1_ref[...]
    ff = jnp.maximum(
        jnp.dot(h1, wf1_ref[...], preferred_element_type=jnp.float32)
        + bf1_ref[...], 0.0)
    u = h1 + jnp.dot(ff, wf2_ref[...],
                     preferred_element_type=jnp.float32) + bf2_ref[...]
    mean2 = jnp.mean(u, axis=0, keepdims=True)
    var2 = jnp.mean((u - mean2) * (u - mean2), axis=0, keepdims=True)
    out_ref[...] = (g2_ref[...] * (u - mean2) * lax.rsqrt(var2 + _EPS)
                    + b2_ref[...])


_post_call = pl.pallas_call(
    _post_body,
    out_shape=jax.ShapeDtypeStruct((_N, _D), jnp.float32),
)


def kernel(x, edge_index, W_gcn, b_gcn, bn1_gamma, bn1_beta,
           W_ff1, b_ff1, W_ff2, b_ff2, bn2_gamma, bn2_beta):
    row = edge_index[0]
    col = edge_index[1]
    pad = _EPAD - _E
    sink = _N + (jnp.arange(pad, dtype=jnp.int32) % (_NPAD - _N))
    spread = jnp.arange(pad, dtype=jnp.int32) * 53 % _N
    rowp = jnp.concatenate([row, spread]).reshape(_NW, _CPW, _K)
    colp = jnp.concatenate([col, sink]).reshape(_NW, _CPW, _K)
    zeros_acc = jnp.zeros((_NPAD, _D), jnp.float32)

    degp = _deg_call(colp, zeros_acc)

    y = _scale_call(x, W_gcn, degp)

    aggp = _agg_call(rowp, colp, y, zeros_acc)

    return _post_call(
        x, y, aggp, degp,
        b_gcn.reshape(1, _D),
        bn1_gamma.reshape(1, _D), bn1_beta.reshape(1, _D),
        W_ff1, b_ff1.reshape(1, 2 * _D),
        W_ff2, b_ff2.reshape(1, _D),
        bn2_gamma.reshape(1, _D), bn2_beta.reshape(1, _D),
    )
